# baseline (speedup 1.0000x reference)
import functools

import jax
import jax.numpy as jnp
from jax import lax
from jax.experimental import pallas as pl
from jax.experimental.pallas import tpu as pltpu
from jax.experimental.pallas import tpu_sc as plsc

N = 10000
G = 100
NPG = 100
D = 128
H = 64
E = 320000
NC = 2
NS = 16
NW = NC * NS
EPW = E // NW
EB = 100
NB = EPW // EB
STRIPE = 624
REM = N - NS * STRIPE

_f32 = jnp.float32


def _seg_sum_body(W, stage_src, y_hbm, srcm_hbm, dstm_hbm, out_hbm,
                  srcb, dstb, rows_a, rows_b, acc, *rest):
    if stage_src:
        ys, sem_a, sem_b, sem_aa, sem_ab = rest
    else:
        (sem_a, sem_b, sem_aa, sem_ab), ys = rest, None
    c = lax.axis_index("c")
    s = lax.axis_index("s")
    wid = c * NS + s

    z = jnp.zeros((16,), _f32)

    def _zero_row(i, _):
        for l in range(W // 16):
            rows_a[i, pl.ds(16 * l, 16)] = z
        return 0

    lax.fori_loop(0, EB, _zero_row, 0)
    base = s * STRIPE
    for off in range(0, STRIPE - EB + 1, EB):
        pltpu.sync_copy(rows_a, acc.at[pl.ds(base + off, EB)])
    rem = STRIPE % EB
    if rem:
        pltpu.sync_copy(rows_a.at[pl.ds(0, rem)],
                        acc.at[pl.ds(base + STRIPE - rem, rem)])

    @pl.when(s == NS - 1)
    def _zero_rem():
        pltpu.sync_copy(rows_a.at[pl.ds(0, REM)],
                        acc.at[pl.ds(NS * STRIPE, REM)])

    if stage_src:
        pltpu.sync_copy(y_hbm.at[pl.ds(base, STRIPE)],
                        ys.at[pl.ds(base, STRIPE)])

        @pl.when(s == NS - 1)
        def _stage_rem():
            pltpu.sync_copy(y_hbm.at[pl.ds(NS * STRIPE, REM)],
                            ys.at[pl.ds(NS * STRIPE, REM)])

    plsc.subcore_barrier()
    ysrc = ys if stage_src else y_hbm

    pltpu.sync_copy(srcm_hbm.at[wid], srcb)
    pltpu.sync_copy(dstm_hbm.at[wid], dstb)

    pltpu.async_copy(ysrc.at[srcb.at[0]], rows_a, sem_a)
    pltpu.async_copy(ysrc.at[srcb.at[1]], rows_b, sem_b)

    def _pair(t, _):
        j0 = 2 * t
        pltpu.make_async_copy(ysrc.at[srcb.at[j0]], rows_a, sem_a).wait()
        pltpu.async_copy(rows_a, acc.at[dstb.at[j0]], sem_aa, add=True)
        pltpu.make_async_copy(ysrc.at[srcb.at[j0 + 1]], rows_b, sem_b).wait()
        pltpu.async_copy(rows_b, acc.at[dstb.at[j0 + 1]], sem_ab, add=True)
        pltpu.make_async_copy(rows_a, acc.at[dstb.at[j0]], sem_aa).wait()
        pltpu.make_async_copy(rows_b, acc.at[dstb.at[j0 + 1]], sem_ab).wait()

        @pl.when(t < NB // 2 - 1)
        def _prefetch():
            pltpu.async_copy(ysrc.at[srcb.at[j0 + 2]], rows_a, sem_a)
            pltpu.async_copy(ysrc.at[srcb.at[j0 + 3]], rows_b, sem_b)

        return 0

    lax.fori_loop(0, NB // 2, _pair, 0)

    plsc.subcore_barrier()
    pltpu.sync_copy(acc.at[pl.ds(base, STRIPE)],
                    out_hbm.at[c].at[pl.ds(base, STRIPE)])

    @pl.when(s == NS - 1)
    def _copy_rem():
        pltpu.sync_copy(acc.at[pl.ds(NS * STRIPE, REM)],
                        out_hbm.at[c].at[pl.ds(NS * STRIPE, REM)])


def _make_seg_sum(W, stage_src):
    scratch = [
        pltpu.VMEM((NB, EB), jnp.int32),
        pltpu.VMEM((NB, EB), jnp.int32),
        pltpu.VMEM((EB, W), _f32),
        pltpu.VMEM((EB, W), _f32),
        pltpu.VMEM_SHARED((N, W), _f32),
    ]
    if stage_src:
        scratch.append(pltpu.VMEM_SHARED((N, W), _f32))
    scratch += [pltpu.SemaphoreType.DMA] * 4
    return pl.kernel(
        functools.partial(_seg_sum_body, W, stage_src),
        out_type=jax.ShapeDtypeStruct((NC, N, W), _f32),
        mesh=plsc.VectorSubcoreMesh(core_axis_name="c", subcore_axis_name="s"),
        compiler_params=pltpu.CompilerParams(use_tc_tiling_on_sc=False),
        scratch_types=scratch,
    )


_seg_sum_d = _make_seg_sum(D, False)
_seg_sum_h = _make_seg_sum(H, True)


def _layer_body(k, partial_ref, xprev_ref, wrel_ref, b_ref, wroot_ref, pw_ref,
                keepg_ref, hp_ref, keep_ref):
    agg = partial_ref[0] + partial_ref[1]
    h = jnp.maximum(
        jnp.dot(agg, wrel_ref[...], preferred_element_type=_f32)
        + b_ref[...][None, :]
        + jnp.dot(xprev_ref[...], wroot_ref[...], preferred_element_type=_f32),
        _f32(0.0))
    pw = pw_ref[...]
    s = (jnp.dot(h, pw, preferred_element_type=_f32)
         / jnp.sqrt(jnp.sum(pw[:, 0] * pw[:, 0])))
    S = jnp.where(keepg_ref[...] > 0, s.reshape(G, NPG), _f32(-1e30))
    gt = (S[:, None, :] > S[:, :, None]).astype(_f32)
    eq = S[:, None, :] == S[:, :, None]
    jlt = (lax.broadcasted_iota(jnp.int32, (1, NPG, NPG), 2)
           < lax.broadcasted_iota(jnp.int32, (1, NPG, NPG), 1))
    rank = jnp.sum(gt + jnp.where(eq & jlt, _f32(1.0), _f32(0.0)), axis=2)
    keep = (rank < k).astype(_f32)
    keep_ref[...] = keep
    hp_ref[...] = h * (jnp.tanh(s) * keep.reshape(N, 1))


def _make_layer(k):
    return pl.pallas_call(
        functools.partial(_layer_body, k),
        out_shape=(jax.ShapeDtypeStruct((N, H), _f32),
                   jax.ShapeDtypeStruct((G, NPG), _f32)),
    )


_layer1 = _make_layer(80)
_layer2 = _make_layer(64)
_layer3 = _make_layer(52)


def _head_body(hp_ref, keep_ref, fc1w_ref, fc1b_ref,
               fc2w_ref, fc2b_ref, fc3w_ref, fc3b_ref, out_ref):
    Hp = hp_ref[...].reshape(G, NPG, H)
    keep3 = keep_ref[...].reshape(G, NPG, 1)
    mean = jnp.sum(Hp, axis=1) / _f32(52.0)
    mx = jnp.max(jnp.where(keep3 > 0, Hp, _f32(-1e30)), axis=1)
    g = jnp.concatenate([mean, mx], axis=1)
    g = jnp.maximum(jnp.dot(g, fc1w_ref[...], preferred_element_type=_f32)
                    + fc1b_ref[...][None, :], _f32(0.0))
    g = jnp.maximum(jnp.dot(g, fc2w_ref[...], preferred_element_type=_f32)
                    + fc2b_ref[...][None, :], _f32(0.0))
    logits = (jnp.dot(g, fc3w_ref[...], preferred_element_type=_f32)
              + fc3b_ref[...][None, :])
    m = jnp.max(logits, axis=1, keepdims=True)
    e = jnp.exp(logits - m)
    out_ref[...] = logits - m - jnp.log(jnp.sum(e, axis=1, keepdims=True))


_head = pl.pallas_call(
    _head_body,
    out_shape=jax.ShapeDtypeStruct((G, 10), _f32),
)


def kernel(x, edge_index, batch, W_rel1, b_rel1, W_root1, pw1, W_rel2,
           b_rel2, W_root2, pw2, W_rel3, b_rel3, W_root3, pw3, fc1_W, fc1_b,
           fc2_W, fc2_b, fc3_W, fc3_b):
    src_main = edge_index[0].reshape(NW, NB, EB)
    dst_main = edge_index[1].reshape(NW, NB, EB)
    edges = (src_main, dst_main)

    p1 = _seg_sum_d(x, *edges)
    hp1, keep1 = _layer1(p1, x, W_rel1, b_rel1, W_root1, pw1.reshape(H, 1),
                         jnp.ones((G, NPG), _f32))
    p2 = _seg_sum_h(hp1, *edges)
    hp2, keep2 = _layer2(p2, hp1, W_rel2, b_rel2, W_root2, pw2.reshape(H, 1),
                         keep1)
    p3 = _seg_sum_h(hp2, *edges)
    hp3, keep3 = _layer3(p3, hp2, W_rel3, b_rel3, W_root3, pw3.reshape(H, 1),
                         keep2)
    return _head(hp3, keep3, fc1_W, fc1_b, fc2_W, fc2_b, fc3_W, fc3_b)


# speedup vs baseline: 1.1837x; 1.1837x over previous
import functools

import jax
import jax.numpy as jnp
from jax import lax
from jax.experimental import pallas as pl
from jax.experimental.pallas import tpu as pltpu
from jax.experimental.pallas import tpu_sc as plsc

N = 10000
G = 100
NPG = 100
D = 128
H = 64
E = 320000
NC = 2
NS = 16
NW = NC * NS
EPW = E // NW
EB = 100
NB = EPW // EB
STRIPE = 624
REM = N - NS * STRIPE

_f32 = jnp.float32


def _seg_sum_body(W, stage_src, y_hbm, srcm_hbm, dstm_hbm, out_hbm,
                  srcb, dstb, rows_a, rows_b, acc, *rest):
    if stage_src:
        ys, sem_a, sem_b = rest
    else:
        (sem_a, sem_b), ys = rest, None
    c = lax.axis_index("c")
    s = lax.axis_index("s")
    wid = c * NS + s

    z = jnp.zeros((16,), _f32)

    def _zero_row(i, _):
        for l in range(W // 16):
            rows_a[i, pl.ds(16 * l, 16)] = z
        return 0

    lax.fori_loop(0, EB, _zero_row, 0)
    base = s * STRIPE
    for off in range(0, STRIPE - EB + 1, EB):
        pltpu.sync_copy(rows_a, acc.at[pl.ds(base + off, EB)])
    rem = STRIPE % EB
    if rem:
        pltpu.sync_copy(rows_a.at[pl.ds(0, rem)],
                        acc.at[pl.ds(base + STRIPE - rem, rem)])

    @pl.when(s == NS - 1)
    def _zero_rem():
        pltpu.sync_copy(rows_a.at[pl.ds(0, REM)],
                        acc.at[pl.ds(NS * STRIPE, REM)])

    if stage_src:
        pltpu.sync_copy(y_hbm.at[pl.ds(base, STRIPE)],
                        ys.at[pl.ds(base, STRIPE)])

        @pl.when(s == NS - 1)
        def _stage_rem():
            pltpu.sync_copy(y_hbm.at[pl.ds(NS * STRIPE, REM)],
                            ys.at[pl.ds(NS * STRIPE, REM)])

    plsc.subcore_barrier()
    ysrc = ys if stage_src else y_hbm

    pltpu.sync_copy(srcm_hbm.at[wid], srcb)
    pltpu.sync_copy(dstm_hbm.at[wid], dstb)

    pltpu.async_copy(ysrc.at[srcb.at[0]], rows_a, sem_a)

    def _pair(t, _):
        j0 = 2 * t
        pltpu.make_async_copy(ysrc.at[srcb.at[j0]], rows_a, sem_a).wait()
        pltpu.async_copy(ysrc.at[srcb.at[j0 + 1]], rows_b, sem_b)
        pltpu.sync_copy(rows_a, acc.at[dstb.at[j0]], add=True)
        pltpu.make_async_copy(ysrc.at[srcb.at[j0 + 1]], rows_b, sem_b).wait()

        @pl.when(t < NB // 2 - 1)
        def _prefetch():
            pltpu.async_copy(ysrc.at[srcb.at[j0 + 2]], rows_a, sem_a)

        pltpu.sync_copy(rows_b, acc.at[dstb.at[j0 + 1]], add=True)
        return 0

    lax.fori_loop(0, NB // 2, _pair, 0)

    plsc.subcore_barrier()
    pltpu.sync_copy(acc.at[pl.ds(base, STRIPE)],
                    out_hbm.at[c].at[pl.ds(base, STRIPE)])

    @pl.when(s == NS - 1)
    def _copy_rem():
        pltpu.sync_copy(acc.at[pl.ds(NS * STRIPE, REM)],
                        out_hbm.at[c].at[pl.ds(NS * STRIPE, REM)])


def _make_seg_sum(W, stage_src):
    scratch = [
        pltpu.VMEM((NB, EB), jnp.int32),
        pltpu.VMEM((NB, EB), jnp.int32),
        pltpu.VMEM((EB, W), _f32),
        pltpu.VMEM((EB, W), _f32),
        pltpu.VMEM_SHARED((N, W), _f32),
    ]
    if stage_src:
        scratch.append(pltpu.VMEM_SHARED((N, W), _f32))
    scratch += [pltpu.SemaphoreType.DMA] * 2
    return pl.kernel(
        functools.partial(_seg_sum_body, W, stage_src),
        out_type=jax.ShapeDtypeStruct((NC, N, W), _f32),
        mesh=plsc.VectorSubcoreMesh(core_axis_name="c", subcore_axis_name="s"),
        compiler_params=pltpu.CompilerParams(use_tc_tiling_on_sc=False),
        scratch_types=scratch,
    )


_seg_sum_d = _make_seg_sum(D, False)
_seg_sum_h = _make_seg_sum(H, True)


def _layer_math(k, partial_ref, xprev_ref, wrel_ref, b_ref, wroot_ref, pw_ref,
                keepg_ref):
    agg = partial_ref[0] + partial_ref[1]
    h = jnp.maximum(
        jnp.dot(agg, wrel_ref[...], preferred_element_type=_f32)
        + b_ref[...][None, :]
        + jnp.dot(xprev_ref[...], wroot_ref[...], preferred_element_type=_f32),
        _f32(0.0))
    pw = pw_ref[...]
    s = (jnp.dot(h, pw, preferred_element_type=_f32)
         / jnp.sqrt(jnp.sum(pw[:, 0] * pw[:, 0])))
    S = jnp.where(keepg_ref[...] > 0, s.reshape(G, NPG), _f32(-1e30))
    gt = (S[:, None, :] > S[:, :, None]).astype(_f32)
    eq = S[:, None, :] == S[:, :, None]
    jlt = (lax.broadcasted_iota(jnp.int32, (1, NPG, NPG), 2)
           < lax.broadcasted_iota(jnp.int32, (1, NPG, NPG), 1))
    rank = jnp.sum(gt + jnp.where(eq & jlt, _f32(1.0), _f32(0.0)), axis=2)
    keep = (rank < k).astype(_f32)
    hp = h * (jnp.tanh(s) * keep.reshape(N, 1))
    return hp, keep


def _layer_body(k, partial_ref, xprev_ref, wrel_ref, b_ref, wroot_ref, pw_ref,
                keepg_ref, hp_ref, keep_ref):
    hp, keep = _layer_math(k, partial_ref, xprev_ref, wrel_ref, b_ref,
                           wroot_ref, pw_ref, keepg_ref)
    keep_ref[...] = keep
    hp_ref[...] = hp


def _make_layer(k):
    return pl.pallas_call(
        functools.partial(_layer_body, k),
        out_shape=(jax.ShapeDtypeStruct((N, H), _f32),
                   jax.ShapeDtypeStruct((G, NPG), _f32)),
    )


_layer1 = _make_layer(80)
_layer2 = _make_layer(64)


def _layer3_head_body(partial_ref, xprev_ref, wrel_ref, b_ref, wroot_ref,
                      pw_ref, keepg_ref, fc1w_ref, fc1b_ref,
                      fc2w_ref, fc2b_ref, fc3w_ref, fc3b_ref, out_ref):
    hp, keep = _layer_math(52, partial_ref, xprev_ref, wrel_ref, b_ref,
                           wroot_ref, pw_ref, keepg_ref)
    Hp = hp.reshape(G, NPG, H)
    keep3 = keep.reshape(G, NPG, 1)
    mean = jnp.sum(Hp, axis=1) / _f32(52.0)
    mx = jnp.max(jnp.where(keep3 > 0, Hp, _f32(-1e30)), axis=1)
    g = jnp.concatenate([mean, mx], axis=1)
    g = jnp.maximum(jnp.dot(g, fc1w_ref[...], preferred_element_type=_f32)
                    + fc1b_ref[...][None, :], _f32(0.0))
    g = jnp.maximum(jnp.dot(g, fc2w_ref[...], preferred_element_type=_f32)
                    + fc2b_ref[...][None, :], _f32(0.0))
    logits = (jnp.dot(g, fc3w_ref[...], preferred_element_type=_f32)
              + fc3b_ref[...][None, :])
    m = jnp.max(logits, axis=1, keepdims=True)
    e = jnp.exp(logits - m)
    out_ref[...] = logits - m - jnp.log(jnp.sum(e, axis=1, keepdims=True))


_layer3_head = pl.pallas_call(
    _layer3_head_body,
    out_shape=jax.ShapeDtypeStruct((G, 10), _f32),
)


def kernel(x, edge_index, batch, W_rel1, b_rel1, W_root1, pw1, W_rel2,
           b_rel2, W_root2, pw2, W_rel3, b_rel3, W_root3, pw3, fc1_W, fc1_b,
           fc2_W, fc2_b, fc3_W, fc3_b):
    src_main = edge_index[0].reshape(NW, NB, EB)
    dst_main = edge_index[1].reshape(NW, NB, EB)
    edges = (src_main, dst_main)

    p1 = _seg_sum_d(x, *edges)
    hp1, keep1 = _layer1(p1, x, W_rel1, b_rel1, W_root1, pw1.reshape(H, 1),
                         jnp.ones((G, NPG), _f32))
    p2 = _seg_sum_h(hp1, *edges)
    hp2, keep2 = _layer2(p2, hp1, W_rel2, b_rel2, W_root2, pw2.reshape(H, 1),
                         keep1)
    p3 = _seg_sum_h(hp2, *edges)
    return _layer3_head(p3, hp2, W_rel3, b_rel3, W_root3, pw3.reshape(H, 1),
                        keep2, fc1_W, fc1_b, fc2_W, fc2_b, fc3_W, fc3_b)
